# baseline (speedup 1.0000x reference)
import numpy as np
import jax
import jax.numpy as jnp
from jax.experimental import pallas as pl

_OUT_CH = 192
_FEAT_DIM = _OUT_CH // 6
_BN = 8192

_TWO_OVER_PI = 0.6366197723675814
_PIO2_HI = np.float32(1.57079637050628662109375)
_PIO2_LO = np.float32(-4.37113900018624283e-8)
_MAGIC = np.float32(1.5 * 2.0**23)

_S1 = np.float32(-1.6666654611e-1)
_S2 = np.float32(8.3321608736e-3)
_S3 = np.float32(-1.9515295891e-4)
_C0 = np.float32(2.443315711809948e-5)
_C1 = np.float32(-1.388731625493765e-3)
_C2 = np.float32(4.166664568298827e-2)


def _sincos(t):
    kb = t * np.float32(_TWO_OVER_PI) + _MAGIC
    bits = jax.lax.bitcast_convert_type(kb, jnp.int32) - np.int32(0x4B400000)
    k = bits.astype(jnp.float32)
    y = t - k * _PIO2_HI
    y = y - k * _PIO2_LO
    z = y * y
    ps = _S3 * z + _S2
    ps = ps * z + _S1
    s = y + (y * z) * ps
    pc = _C0 * z + _C1
    pc = pc * z + _C2
    c = (z * z) * pc + (np.float32(1.0) - np.float32(0.5) * z)
    swap = (bits & 1) == 1
    sin_base = jnp.where(swap, c, s)
    cos_base = jnp.where(swap, s, c)
    sin_flip = (bits & 2) << 30
    cos_flip = ((bits + 1) & 2) << 30
    sin_t = jax.lax.bitcast_convert_type(
        jax.lax.bitcast_convert_type(sin_base, jnp.int32) ^ sin_flip, jnp.float32)
    cos_t = jax.lax.bitcast_convert_type(
        jax.lax.bitcast_convert_type(cos_base, jnp.int32) ^ cos_flip, jnp.float32)
    return sin_t, cos_t


def _pe_kernel(s_ref, r_ref, o_ref):
    s = s_ref[...]
    fd = _FEAT_DIM
    for i in range(3):
        t = r_ref[:, i : i + 1, :] * s
        sin_t, cos_t = _sincos(t)
        o_ref[:, 2 * i * fd : (2 * i + 1) * fd, :] = sin_t
        o_ref[:, (2 * i + 1) * fd : (2 * i + 2) * fd, :] = cos_t


def kernel(re_xyz, x):
    B, _, npoint, nsample = re_xyz.shape
    C = x.shape[1]
    N = npoint * nsample
    r = re_xyz.reshape(B, 3, N)

    fr = jnp.arange(_FEAT_DIM, dtype=jnp.float32)
    dim_mat = jnp.power(jnp.float32(500.0), (1.0 / _FEAT_DIM) * fr)
    scale = (50.0 / dim_mat).reshape(1, _FEAT_DIM, 1)

    nblk = N // _BN
    pe = pl.pallas_call(
        _pe_kernel,
        grid=(B, nblk),
        in_specs=[
            pl.BlockSpec((1, _FEAT_DIM, 1), lambda b, n: (0, 0, 0)),
            pl.BlockSpec((1, 3, _BN), lambda b, n: (b, 0, n)),
        ],
        out_specs=pl.BlockSpec((1, C, _BN), lambda b, n: (b, 0, n)),
        out_shape=jax.ShapeDtypeStruct((B, C, N), jnp.float32),
    )(scale, r)
    pe4 = pe.reshape(B, C, npoint, nsample)
    return x * pe4 + pe4
